# baseline (speedup 1.0000x reference)
import functools

import jax
import jax.numpy as jnp
from jax.experimental import pallas as pl


_ROWS = 32768


def _probe_body(x_ref, o_ref):
    o_ref[...] = x_ref[:256, :16]


@functools.partial(jax.jit, static_argnames=())
def kernel(x, W, b):
    B, S, D = x.shape
    E = W.shape[0]
    n_tokens = B * S
    xv = x.reshape(n_tokens // 2, D * 2)
    n_steps = xv.shape[0] // 1024
    out = pl.pallas_call(
        _probe_body,
        grid=(n_steps,),
        in_specs=[pl.BlockSpec((1024, D * 2), lambda i: (i, 0))],
        out_specs=pl.BlockSpec((256, E), lambda i: (i, 0)),
        out_shape=jax.ShapeDtypeStruct((n_steps * 256, E), jnp.float32),
    )(xv)
    disp = jnp.zeros((B, S, E), jnp.float32) + out[0, 0]
    return (disp, disp, jnp.zeros((E,), jnp.float32))


# speedup vs baseline: 3.2076x; 3.2076x over previous
import functools

import jax
import jax.numpy as jnp
from jax.experimental import pallas as pl


_TILE = 2048


def _router_body(x_ref, wt_ref, b_ref, disp_ref, cnt_ref):
    logits = jnp.dot(
        x_ref[...], wt_ref[...],
        preferred_element_type=jnp.float32,
    )
    lt = logits.T + b_ref[...]
    m = jnp.max(lt, axis=0, keepdims=True)
    e = jnp.exp(lt - m)
    scores = e / jnp.sum(e, axis=0, keepdims=True)
    idx = jax.lax.broadcasted_iota(jnp.int32, scores.shape, 0)
    m1 = jnp.max(scores, axis=0, keepdims=True)
    i1 = jnp.min(jnp.where(scores == m1, idx, 16), axis=0, keepdims=True)
    mask1 = idx == i1
    rest = jnp.where(mask1, -1.0, scores)
    m2 = jnp.max(rest, axis=0, keepdims=True)
    i2 = jnp.min(jnp.where(rest == m2, idx, 16), axis=0, keepdims=True)
    disp_t = jnp.where(mask1 | (idx == i2), scores, 0.0)
    disp_ref[...] = disp_t.T

    @pl.when(pl.program_id(0) == 0)
    def _init():
        cnt_ref[...] = jnp.zeros_like(cnt_ref)

    cnt_ref[...] += jnp.sum(disp_t, axis=1, keepdims=True)


@functools.partial(jax.jit, static_argnames=())
def kernel(x, W, b):
    B, S, D = x.shape
    E = W.shape[0]
    n_tokens = B * S
    xf = x.reshape(n_tokens, D)
    wt = W.T
    bc = b.reshape(E, 1)
    grid = (n_tokens // _TILE,)
    disp, cnt = pl.pallas_call(
        _router_body,
        grid=grid,
        in_specs=[
            pl.BlockSpec((_TILE, D), lambda i: (i, 0)),
            pl.BlockSpec((D, E), lambda i: (0, 0)),
            pl.BlockSpec((E, 1), lambda i: (0, 0)),
        ],
        out_specs=[
            pl.BlockSpec((_TILE, E), lambda i: (i, 0)),
            pl.BlockSpec((E, 1), lambda i: (0, 0)),
        ],
        out_shape=[
            jax.ShapeDtypeStruct((n_tokens, E), jnp.float32),
            jax.ShapeDtypeStruct((E, 1), jnp.float32),
        ],
    )(xf, wt, bc)
    dispatch = disp.reshape(B, S, E)
    return (dispatch, dispatch, cnt.reshape(E))
